# Initial kernel scaffold; baseline (speedup 1.0000x reference)
import jax
import jax.numpy as jnp
from jax.experimental import pallas as pl


def kernel(bert_embedding, features, row_ids, node_embedding, edge_index, W_self, W_neigh, b_sage, W1, b1, W2, b2, W3, b3):
    raise NotImplementedError("write your pallas kernel here")


# speedup vs baseline: 15.8552x; 15.8552x over previous
import functools

import jax
import jax.numpy as jnp
from jax import lax
from jax.experimental import pallas as pl
from jax.experimental.pallas import tpu as pltpu
from jax.experimental.pallas import tpu_sc as plsc

NC = 2
NS = 16
NW = NC * NS
CH = 64


GRP = 4


def _sc_aggregate(node_embedding, edge3, rid3):
    N = node_embedding.shape[0]
    NP = ((N + NS * 8 - 1) // (NS * 8)) * (NS * 8)
    TOT = edge3.shape[1]
    RCH = rid3.shape[0]
    R = RCH * CH
    NG = TOT // GRP
    GPW = pl.cdiv(NG, NW)
    ROWS_PT = NP // NS
    TPG = RCH // (NS * GRP)

    f32 = jnp.float32
    aug = jnp.pad(
        jnp.concatenate(
            [node_embedding, jnp.ones((N, 1), f32), jnp.zeros((N, 7), f32)],
            axis=1),
        ((0, NP - N), (0, 0)))
    zeros16 = jnp.zeros((ROWS_PT, 16), f32)

    mesh = plsc.VectorSubcoreMesh(
        core_axis_name="c", subcore_axis_name="s",
        num_cores=NC, num_subcores=NS)

    @functools.partial(
        pl.kernel,
        out_type=[
            jax.ShapeDtypeStruct((NC, R, 16), f32),
            jax.ShapeDtypeStruct((R, 16), f32),
        ],
        mesh=mesh,
        compiler_params=pltpu.CompilerParams(use_tc_tiling_on_sc=False),
        scratch_types=[
            pltpu.VMEM_SHARED((NP, 16), f32),
            pltpu.VMEM((GRP, CH), jnp.int32),
            pltpu.VMEM((GRP, CH), jnp.int32),
            pltpu.VMEM((CH, 16), f32),
            pltpu.SemaphoreType.DMA,
        ],
    )
    def k(aug_hbm, edge_hbm, rid_hbm, z_hbm,
          sel, nselo,
          acc_sh, sbuf, dbuf, rows, sem):
        cid = lax.axis_index("c")
        sid = lax.axis_index("s")
        w = sid * NC + cid

        r0 = sid * ROWS_PT
        pltpu.sync_copy(z_hbm, acc_sh.at[pl.ds(r0, ROWS_PT)])
        plsc.subcore_barrier()

        g_lo = w * GPW
        g_hi = jnp.minimum(g_lo + GPW, NG)

        def body(g, carry):
            pltpu.sync_copy(edge_hbm.at[0, pl.ds(g * GRP, GRP)], sbuf)
            pltpu.sync_copy(edge_hbm.at[1, pl.ds(g * GRP, GRP)], dbuf)
            for j in range(GRP):
                pltpu.async_copy(aug_hbm.at[sbuf.at[j]], rows, sem).wait()
                pltpu.sync_copy(rows, acc_sh.at[dbuf.at[j]], add=True)
            return carry

        lax.fori_loop(g_lo, g_hi, body, 0)
        plsc.subcore_barrier()

        for t in range(TPG):
            g = sid * TPG + t
            pltpu.sync_copy(rid_hbm.at[pl.ds(g * GRP, GRP)], sbuf)
            for j in range(GRP):
                base = (g * GRP + j) * CH
                pltpu.async_copy(acc_sh.at[sbuf.at[j]], rows, sem).wait()
                pltpu.sync_copy(rows, sel.at[cid, pl.ds(base, CH)])

                @pl.when(cid == (t % NC))
                def _():
                    pltpu.async_copy(aug_hbm.at[sbuf.at[j]], rows, sem).wait()
                    pltpu.sync_copy(rows, nselo.at[pl.ds(base, CH)])

    return k(aug, edge3, rid3, zeros16)


def _tc_head(bert, W1t, b1r, sel, nsel,
             Wst, Wnt, bsr, W2a, W2b, W2c, b2r, W3p, b3p):
    Bn = bert.shape[0]
    BLK = 1024
    H = Bn // BLK

    def body(bert_ref, W1t_ref, b1_ref, sh_ref, st_ref,
             nh_ref, nt_ref, Wst_ref, Wnt_ref, bs_ref,
             W2a_ref, W2b_ref, W2c_ref, b2_ref, W3p_ref, b3_ref, out_ref):
        f32 = jnp.float32
        x1 = jnp.dot(bert_ref[...], W1t_ref[...], preferred_element_type=f32)
        x1 = jnp.maximum(x1 + b1_ref[...], 0.0)
        sh = sh_ref[0] + sh_ref[1]
        st = st_ref[0] + st_ref[1]
        hh = sh[:, 0:8] / jnp.maximum(sh[:, 8:9], 1.0)
        ht = st[:, 0:8] / jnp.maximum(st[:, 8:9], 1.0)
        rel_h = (jnp.dot(nh_ref[:, 0:8], Wst_ref[...], preferred_element_type=f32)
                 + jnp.dot(hh, Wnt_ref[...], preferred_element_type=f32)
                 + bs_ref[...])
        rel_t = (jnp.dot(nt_ref[:, 0:8], Wst_ref[...], preferred_element_type=f32)
                 + jnp.dot(ht, Wnt_ref[...], preferred_element_type=f32)
                 + bs_ref[...])
        y = (jnp.dot(x1, W2a_ref[...], preferred_element_type=f32)
             + jnp.dot(rel_h, W2b_ref[...], preferred_element_type=f32)
             + jnp.dot(rel_t, W2c_ref[...], preferred_element_type=f32)
             + b2_ref[...])
        y = jnp.maximum(y, 0.0)
        out_ref[...] = jnp.dot(y, W3p_ref[...], preferred_element_type=f32) + b3_ref[...]

    grid = (Bn // BLK,)
    return pl.pallas_call(
        body,
        grid=grid,
        in_specs=[
            pl.BlockSpec((BLK, 768), lambda i: (i, 0)),
            pl.BlockSpec((768, 128), lambda i: (0, 0)),
            pl.BlockSpec((1, 128), lambda i: (0, 0)),
            pl.BlockSpec((2, BLK, 16), lambda i: (0, i, 0)),
            pl.BlockSpec((2, BLK, 16), lambda i: (0, H + i, 0)),
            pl.BlockSpec((BLK, 16), lambda i: (i, 0)),
            pl.BlockSpec((BLK, 16), lambda i: (H + i, 0)),
            pl.BlockSpec((8, 8), lambda i: (0, 0)),
            pl.BlockSpec((8, 8), lambda i: (0, 0)),
            pl.BlockSpec((1, 8), lambda i: (0, 0)),
            pl.BlockSpec((128, 16), lambda i: (0, 0)),
            pl.BlockSpec((8, 16), lambda i: (0, 0)),
            pl.BlockSpec((8, 16), lambda i: (0, 0)),
            pl.BlockSpec((1, 16), lambda i: (0, 0)),
            pl.BlockSpec((16, 8), lambda i: (0, 0)),
            pl.BlockSpec((1, 8), lambda i: (0, 0)),
        ],
        out_specs=pl.BlockSpec((BLK, 8), lambda i: (i, 0)),
        out_shape=jax.ShapeDtypeStruct((Bn, 8), jnp.float32),
    )(bert, W1t, b1r, sel, sel, nsel, nsel,
      Wst, Wnt, bsr, W2a, W2b, W2c, b2r, W3p, b3p)


def kernel(bert_embedding, features, row_ids, node_embedding, edge_index,
           W_self, W_neigh, b_sage, W1, b1, W2, b2, W3, b3):
    del features
    E = edge_index.shape[1]
    Bn = bert_embedding.shape[0]

    edge3 = edge_index.astype(jnp.int32).reshape(2, E // CH, CH)
    rid = row_ids.astype(jnp.int32).reshape(-1)
    rid3 = rid.reshape(rid.shape[0] // CH, CH)

    sel, nsel = _sc_aggregate(node_embedding, edge3, rid3)

    W1t = W1.T
    b1r = b1.reshape(1, 128)
    Wst = W_self.T
    Wnt = W_neigh.T
    bsr = b_sage.reshape(1, 8)
    W2a = W2[:, :128].T
    W2b = W2[:, 128:136].T
    W2c = W2[:, 136:144].T
    b2r = b2.reshape(1, 16)
    W3p = jnp.pad(W3.T, ((0, 0), (0, 5)))
    b3p = jnp.pad(b3, (0, 5)).reshape(1, 8)

    out8 = _tc_head(bert_embedding, W1t, b1r, sel, nsel,
                    Wst, Wnt, bsr, W2a, W2b, W2c, b2r, W3p, b3p)
    return out8[:, :3]


# speedup vs baseline: 28.3719x; 1.7894x over previous
import functools

import jax
import jax.numpy as jnp
from jax import lax
from jax.experimental import pallas as pl
from jax.experimental.pallas import tpu as pltpu
from jax.experimental.pallas import tpu_sc as plsc

NC = 2
NS = 16
NW = NC * NS
CH = 128


GRP = 8


def _sc_aggregate(node_embedding, edge3, rid3):
    N = node_embedding.shape[0]
    NP = ((N + NS * 8 - 1) // (NS * 8)) * (NS * 8)
    TOT = edge3.shape[1]
    RCH = rid3.shape[0]
    R = RCH * CH
    NG = TOT // GRP
    GPW = pl.cdiv(NG, NW)
    ROWS_PT = NP // NS
    TPG = RCH // (NS * GRP)

    f32 = jnp.float32
    ne_pad = jnp.pad(node_embedding, ((0, NP - N), (0, 0)))
    zeros8 = jnp.zeros((ROWS_PT, 8), f32)
    ones8 = jnp.ones((CH, 8), f32)

    mesh = plsc.VectorSubcoreMesh(
        core_axis_name="c", subcore_axis_name="s",
        num_cores=NC, num_subcores=NS)

    @functools.partial(
        pl.kernel,
        out_type=[
            jax.ShapeDtypeStruct((NC, R, 8), f32),
            jax.ShapeDtypeStruct((NC, R, 8), f32),
            jax.ShapeDtypeStruct((R, 8), f32),
        ],
        mesh=mesh,
        compiler_params=pltpu.CompilerParams(use_tc_tiling_on_sc=False),
        scratch_types=[
            pltpu.VMEM_SHARED((NP, 8), f32),
            pltpu.VMEM((GRP, CH), jnp.int32),
            pltpu.VMEM((GRP, CH), jnp.int32),
            pltpu.VMEM((CH, 8), f32),
            pltpu.VMEM((CH, 8), f32),
            pltpu.SemaphoreType.DMA,
            pltpu.SemaphoreType.DMA,
            pltpu.SemaphoreType.DMA,
            pltpu.SemaphoreType.DMA,
        ],
    )
    def k(ne_hbm, edge_hbm, rid_hbm, z_hbm, o_hbm,
          aggsel, acc2sel, nselo,
          acc_sh, sbuf, dbuf, rows0, rows1, sg0, sg1, ss0, ss1):
        cid = lax.axis_index("c")
        sid = lax.axis_index("s")
        w = sid * NC + cid
        rows = (rows0, rows1)
        semg = (sg0, sg1)
        sems = (ss0, ss1)

        r0 = sid * ROWS_PT
        pltpu.sync_copy(z_hbm, acc_sh.at[pl.ds(r0, ROWS_PT)])
        plsc.subcore_barrier()

        g_lo = w * GPW
        g_hi = jnp.minimum(g_lo + GPW, NG)

        def bodyA(g, carry):
            pltpu.sync_copy(edge_hbm.at[0, pl.ds(g * GRP, GRP)], sbuf)
            pltpu.sync_copy(edge_hbm.at[1, pl.ds(g * GRP, GRP)], dbuf)
            gd = [None] * GRP
            sd = [None] * GRP
            gd[0] = pltpu.async_copy(ne_hbm.at[sbuf.at[0]], rows0, sg0)
            for j in range(GRP):
                gd[j].wait()
                sd[j] = pltpu.async_copy(
                    rows[j % 2], acc_sh.at[dbuf.at[j]], sems[j % 2], add=True)
                if j + 1 < GRP:
                    if j >= 1:
                        sd[j - 1].wait()
                    gd[j + 1] = pltpu.async_copy(
                        ne_hbm.at[sbuf.at[j + 1]], rows[(j + 1) % 2],
                        semg[(j + 1) % 2])
            sd[GRP - 2].wait()
            sd[GRP - 1].wait()
            return carry

        lax.fori_loop(g_lo, g_hi, bodyA, 0)
        plsc.subcore_barrier()

        def select(out_ref, src_table, also_nsel):
            for t in range(TPG):
                g = sid * TPG + t
                pltpu.sync_copy(rid_hbm.at[pl.ds(g * GRP, GRP)], sbuf)
                for j in range(GRP):
                    base = (g * GRP + j) * CH
                    pltpu.async_copy(src_table.at[sbuf.at[j]], rows0, sg0).wait()
                    pltpu.sync_copy(rows0, out_ref.at[cid, pl.ds(base, CH)])
                    if also_nsel:
                        @pl.when(cid == (t % NC))
                        def _():
                            pltpu.async_copy(
                                ne_hbm.at[sbuf.at[j]], rows1, sg1).wait()
                            pltpu.sync_copy(rows1, nselo.at[pl.ds(base, CH)])

        select(aggsel, acc_sh, True)
        plsc.subcore_barrier()

        pltpu.sync_copy(o_hbm, rows0)

        def bodyB(g, carry):
            pltpu.sync_copy(edge_hbm.at[1, pl.ds(g * GRP, GRP)], dbuf)
            sd = [pltpu.async_copy(
                      rows0, acc_sh.at[dbuf.at[j]], sems[j % 2], add=True)
                  for j in range(GRP)]
            for d in sd:
                d.wait()
            return carry

        lax.fori_loop(g_lo, g_hi, bodyB, 0)
        plsc.subcore_barrier()

        select(acc2sel, acc_sh, False)

    return k(ne_pad, edge3, rid3, zeros8, ones8)


def _tc_head(bert, W1t, b1r, aggsel, acc2sel, nsel,
             Wst, Wnt, bsr, W2a, W2b, W2c, b2r, W3p, b3p):
    Bn = bert.shape[0]
    BLK = 1024
    H = Bn // BLK

    def body(bert_ref, W1t_ref, b1_ref, ah_ref, at_ref, ch_ref, ct_ref,
             nh_ref, nt_ref, Wst_ref, Wnt_ref, bs_ref,
             W2a_ref, W2b_ref, W2c_ref, b2_ref, W3p_ref, b3_ref, out_ref):
        f32 = jnp.float32
        x1 = jnp.dot(bert_ref[...], W1t_ref[...], preferred_element_type=f32)
        x1 = jnp.maximum(x1 + b1_ref[...], 0.0)
        ah = ah_ref[0] + ah_ref[1]
        at = at_ref[0] + at_ref[1]
        dh = (ch_ref[0, :, 0:1] + ch_ref[1, :, 0:1]) - ah[:, 0:1]
        dt = (ct_ref[0, :, 0:1] + ct_ref[1, :, 0:1]) - at[:, 0:1]
        hh = ah / jnp.maximum(dh, 1.0)
        ht = at / jnp.maximum(dt, 1.0)
        rel_h = (jnp.dot(nh_ref[...], Wst_ref[...], preferred_element_type=f32)
                 + jnp.dot(hh, Wnt_ref[...], preferred_element_type=f32)
                 + bs_ref[...])
        rel_t = (jnp.dot(nt_ref[...], Wst_ref[...], preferred_element_type=f32)
                 + jnp.dot(ht, Wnt_ref[...], preferred_element_type=f32)
                 + bs_ref[...])
        y = (jnp.dot(x1, W2a_ref[...], preferred_element_type=f32)
             + jnp.dot(rel_h, W2b_ref[...], preferred_element_type=f32)
             + jnp.dot(rel_t, W2c_ref[...], preferred_element_type=f32)
             + b2_ref[...])
        y = jnp.maximum(y, 0.0)
        out_ref[...] = jnp.dot(y, W3p_ref[...], preferred_element_type=f32) + b3_ref[...]

    grid = (Bn // BLK,)
    return pl.pallas_call(
        body,
        grid=grid,
        in_specs=[
            pl.BlockSpec((BLK, 768), lambda i: (i, 0)),
            pl.BlockSpec((768, 128), lambda i: (0, 0)),
            pl.BlockSpec((1, 128), lambda i: (0, 0)),
            pl.BlockSpec((2, BLK, 8), lambda i: (0, i, 0)),
            pl.BlockSpec((2, BLK, 8), lambda i: (0, H + i, 0)),
            pl.BlockSpec((2, BLK, 8), lambda i: (0, i, 0)),
            pl.BlockSpec((2, BLK, 8), lambda i: (0, H + i, 0)),
            pl.BlockSpec((BLK, 8), lambda i: (i, 0)),
            pl.BlockSpec((BLK, 8), lambda i: (H + i, 0)),
            pl.BlockSpec((8, 8), lambda i: (0, 0)),
            pl.BlockSpec((8, 8), lambda i: (0, 0)),
            pl.BlockSpec((1, 8), lambda i: (0, 0)),
            pl.BlockSpec((128, 16), lambda i: (0, 0)),
            pl.BlockSpec((8, 16), lambda i: (0, 0)),
            pl.BlockSpec((8, 16), lambda i: (0, 0)),
            pl.BlockSpec((1, 16), lambda i: (0, 0)),
            pl.BlockSpec((16, 8), lambda i: (0, 0)),
            pl.BlockSpec((1, 8), lambda i: (0, 0)),
        ],
        out_specs=pl.BlockSpec((BLK, 8), lambda i: (i, 0)),
        out_shape=jax.ShapeDtypeStruct((Bn, 8), jnp.float32),
    )(bert, W1t, b1r, aggsel, aggsel, acc2sel, acc2sel, nsel, nsel,
      Wst, Wnt, bsr, W2a, W2b, W2c, b2r, W3p, b3p)


def kernel(bert_embedding, features, row_ids, node_embedding, edge_index,
           W_self, W_neigh, b_sage, W1, b1, W2, b2, W3, b3):
    del features
    E = edge_index.shape[1]
    Bn = bert_embedding.shape[0]

    edge3 = edge_index.astype(jnp.int32).reshape(2, E // CH, CH)
    rid = row_ids.astype(jnp.int32).reshape(-1)
    rid3 = rid.reshape(rid.shape[0] // CH, CH)

    aggsel, acc2sel, nsel = _sc_aggregate(node_embedding, edge3, rid3)

    W1t = W1.T
    b1r = b1.reshape(1, 128)
    Wst = W_self.T
    Wnt = W_neigh.T
    bsr = b_sage.reshape(1, 8)
    W2a = W2[:, :128].T
    W2b = W2[:, 128:136].T
    W2c = W2[:, 136:144].T
    b2r = b2.reshape(1, 16)
    W3p = jnp.pad(W3.T, ((0, 0), (0, 5)))
    b3p = jnp.pad(b3, (0, 5)).reshape(1, 8)

    out8 = _tc_head(bert_embedding, W1t, b1r, aggsel, acc2sel, nsel,
                    Wst, Wnt, bsr, W2a, W2b, W2c, b2r, W3p, b3p)
    return out8[:, :3]


# speedup vs baseline: 29.9262x; 1.0548x over previous
import functools

import jax
import jax.numpy as jnp
from jax import lax
from jax.experimental import pallas as pl
from jax.experimental.pallas import tpu as pltpu
from jax.experimental.pallas import tpu_sc as plsc

NC = 2
NS = 16
NW = NC * NS
CH = 128


GRP = 8


def _sc_aggregate(node_embedding, edge3, rid3):
    N = node_embedding.shape[0]
    NP = ((N + NS * 8 - 1) // (NS * 8)) * (NS * 8)
    TOT = edge3.shape[1]
    RCH = rid3.shape[0]
    R = RCH * CH
    NG = TOT // GRP
    GPW = pl.cdiv(NG, NW)
    ROWS_PT = NP // NS
    TPG = RCH // (NS * GRP)

    f32 = jnp.float32
    ne_pad = jnp.pad(node_embedding, ((0, NP - N), (0, 0)))
    zeros8 = jnp.zeros((ROWS_PT, 8), f32)
    ones8 = jnp.ones((CH, 8), f32)

    mesh = plsc.VectorSubcoreMesh(
        core_axis_name="c", subcore_axis_name="s",
        num_cores=NC, num_subcores=NS)

    @functools.partial(
        pl.kernel,
        out_type=[
            jax.ShapeDtypeStruct((NC, R, 8), f32),
            jax.ShapeDtypeStruct((NC, R, 8), f32),
            jax.ShapeDtypeStruct((R, 8), f32),
        ],
        mesh=mesh,
        compiler_params=pltpu.CompilerParams(use_tc_tiling_on_sc=False),
        scratch_types=[
            pltpu.VMEM_SHARED((NP, 8), f32),
            pltpu.VMEM((2, GRP, CH), jnp.int32),
            pltpu.VMEM((CH, 8), f32),
            pltpu.VMEM((CH, 8), f32),
            pltpu.SemaphoreType.DMA,
            pltpu.SemaphoreType.DMA,
            pltpu.SemaphoreType.DMA,
            pltpu.SemaphoreType.DMA,
        ],
    )
    def k(ne_hbm, edge_hbm, rid_hbm, z_hbm, o_hbm,
          aggsel, acc2sel, nselo,
          acc_sh, ibuf, rows0, rows1, sg0, sg1, ss0, ss1):
        cid = lax.axis_index("c")
        sid = lax.axis_index("s")
        w = sid * NC + cid
        rows = (rows0, rows1)
        semg = (sg0, sg1)
        sems = (ss0, ss1)

        r0 = sid * ROWS_PT
        pltpu.sync_copy(z_hbm, acc_sh.at[pl.ds(r0, ROWS_PT)])
        plsc.subcore_barrier()

        g_lo = w * GPW
        g_hi = jnp.minimum(g_lo + GPW, NG)

        def bodyA(g, carry):
            pltpu.sync_copy(edge_hbm.at[:, pl.ds(g * GRP, GRP)], ibuf)
            gd = [None] * GRP
            sd = [None] * GRP
            gd[0] = pltpu.async_copy(ne_hbm.at[ibuf.at[0, 0]], rows0, sg0)
            for j in range(GRP):
                gd[j].wait()
                sd[j] = pltpu.async_copy(
                    rows[j % 2], acc_sh.at[ibuf.at[1, j]], sems[j % 2], add=True)
                if j + 1 < GRP:
                    if j >= 1:
                        sd[j - 1].wait()
                    gd[j + 1] = pltpu.async_copy(
                        ne_hbm.at[ibuf.at[0, j + 1]], rows[(j + 1) % 2],
                        semg[(j + 1) % 2])
            sd[GRP - 2].wait()
            sd[GRP - 1].wait()
            return carry

        lax.fori_loop(g_lo, g_hi, bodyA, 0)
        plsc.subcore_barrier()

        def select(out_ref, src_table, also_nsel):
            for t in range(TPG):
                g = sid * TPG + t
                pltpu.sync_copy(rid_hbm.at[pl.ds(g * GRP, GRP)], ibuf.at[0])
                for j in range(GRP):
                    base = (g * GRP + j) * CH
                    pltpu.async_copy(src_table.at[ibuf.at[0, j]], rows0, sg0).wait()
                    pltpu.sync_copy(rows0, out_ref.at[cid, pl.ds(base, CH)])
                    if also_nsel:
                        @pl.when(cid == (t % NC))
                        def _():
                            pltpu.async_copy(
                                ne_hbm.at[ibuf.at[0, j]], rows1, sg1).wait()
                            pltpu.sync_copy(rows1, nselo.at[pl.ds(base, CH)])

        select(aggsel, acc_sh, True)
        plsc.subcore_barrier()

        pltpu.sync_copy(o_hbm, rows0)

        def bodyB(g, carry):
            pltpu.sync_copy(edge_hbm.at[1, pl.ds(g * GRP, GRP)], ibuf.at[1])
            sd = [pltpu.async_copy(
                      rows0, acc_sh.at[ibuf.at[1, j]], sems[j % 2], add=True)
                  for j in range(GRP)]
            for d in sd:
                d.wait()
            return carry

        lax.fori_loop(g_lo, g_hi, bodyB, 0)
        plsc.subcore_barrier()

        select(acc2sel, acc_sh, False)

    return k(ne_pad, edge3, rid3, zeros8, ones8)


def _tc_head(bert, W1t, b1r, aggsel, acc2sel, nsel,
             Wst, Wnt, bsr, W2a, W2b, W2c, b2r, W3p, b3p):
    Bn = bert.shape[0]
    BLK = 1024
    H = Bn // BLK

    def body(bert_ref, W1t_ref, b1_ref, ah_ref, at_ref, ch_ref, ct_ref,
             nh_ref, nt_ref, Wst_ref, Wnt_ref, bs_ref,
             W2a_ref, W2b_ref, W2c_ref, b2_ref, W3p_ref, b3_ref, out_ref):
        f32 = jnp.float32
        x1 = jnp.dot(bert_ref[...], W1t_ref[...], preferred_element_type=f32)
        x1 = jnp.maximum(x1 + b1_ref[...], 0.0)
        ah = ah_ref[0] + ah_ref[1]
        at = at_ref[0] + at_ref[1]
        dh = (ch_ref[0, :, 0:1] + ch_ref[1, :, 0:1]) - ah[:, 0:1]
        dt = (ct_ref[0, :, 0:1] + ct_ref[1, :, 0:1]) - at[:, 0:1]
        hh = ah / jnp.maximum(dh, 1.0)
        ht = at / jnp.maximum(dt, 1.0)
        rel_h = (jnp.dot(nh_ref[...], Wst_ref[...], preferred_element_type=f32)
                 + jnp.dot(hh, Wnt_ref[...], preferred_element_type=f32)
                 + bs_ref[...])
        rel_t = (jnp.dot(nt_ref[...], Wst_ref[...], preferred_element_type=f32)
                 + jnp.dot(ht, Wnt_ref[...], preferred_element_type=f32)
                 + bs_ref[...])
        y = (jnp.dot(x1, W2a_ref[...], preferred_element_type=f32)
             + jnp.dot(rel_h, W2b_ref[...], preferred_element_type=f32)
             + jnp.dot(rel_t, W2c_ref[...], preferred_element_type=f32)
             + b2_ref[...])
        y = jnp.maximum(y, 0.0)
        out_ref[...] = jnp.dot(y, W3p_ref[...], preferred_element_type=f32) + b3_ref[...]

    grid = (Bn // BLK,)
    return pl.pallas_call(
        body,
        grid=grid,
        in_specs=[
            pl.BlockSpec((BLK, 768), lambda i: (i, 0)),
            pl.BlockSpec((768, 128), lambda i: (0, 0)),
            pl.BlockSpec((1, 128), lambda i: (0, 0)),
            pl.BlockSpec((2, BLK, 8), lambda i: (0, i, 0)),
            pl.BlockSpec((2, BLK, 8), lambda i: (0, H + i, 0)),
            pl.BlockSpec((2, BLK, 8), lambda i: (0, i, 0)),
            pl.BlockSpec((2, BLK, 8), lambda i: (0, H + i, 0)),
            pl.BlockSpec((BLK, 8), lambda i: (i, 0)),
            pl.BlockSpec((BLK, 8), lambda i: (H + i, 0)),
            pl.BlockSpec((8, 8), lambda i: (0, 0)),
            pl.BlockSpec((8, 8), lambda i: (0, 0)),
            pl.BlockSpec((1, 8), lambda i: (0, 0)),
            pl.BlockSpec((128, 16), lambda i: (0, 0)),
            pl.BlockSpec((8, 16), lambda i: (0, 0)),
            pl.BlockSpec((8, 16), lambda i: (0, 0)),
            pl.BlockSpec((1, 16), lambda i: (0, 0)),
            pl.BlockSpec((16, 8), lambda i: (0, 0)),
            pl.BlockSpec((1, 8), lambda i: (0, 0)),
        ],
        out_specs=pl.BlockSpec((BLK, 8), lambda i: (i, 0)),
        out_shape=jax.ShapeDtypeStruct((Bn, 8), jnp.float32),
    )(bert, W1t, b1r, aggsel, aggsel, acc2sel, acc2sel, nsel, nsel,
      Wst, Wnt, bsr, W2a, W2b, W2c, b2r, W3p, b3p)


def kernel(bert_embedding, features, row_ids, node_embedding, edge_index,
           W_self, W_neigh, b_sage, W1, b1, W2, b2, W3, b3):
    del features
    E = edge_index.shape[1]
    Bn = bert_embedding.shape[0]

    edge3 = edge_index.astype(jnp.int32).reshape(2, E // CH, CH)
    rid = row_ids.astype(jnp.int32).reshape(-1)
    rid3 = rid.reshape(rid.shape[0] // CH, CH)

    aggsel, acc2sel, nsel = _sc_aggregate(node_embedding, edge3, rid3)

    W1t = W1.T
    b1r = b1.reshape(1, 128)
    Wst = W_self.T
    Wnt = W_neigh.T
    bsr = b_sage.reshape(1, 8)
    W2a = W2[:, :128].T
    W2b = W2[:, 128:136].T
    W2c = W2[:, 136:144].T
    b2r = b2.reshape(1, 16)
    W3p = jnp.pad(W3.T, ((0, 0), (0, 5)))
    b3p = jnp.pad(b3, (0, 5)).reshape(1, 8)

    out8 = _tc_head(bert_embedding, W1t, b1r, aggsel, acc2sel, nsel,
                    Wst, Wnt, bsr, W2a, W2b, W2c, b2r, W3p, b3p)
    return out8[:, :3]


# speedup vs baseline: 29.9774x; 1.0017x over previous
import functools

import jax
import jax.numpy as jnp
from jax import lax
from jax.experimental import pallas as pl
from jax.experimental.pallas import tpu as pltpu
from jax.experimental.pallas import tpu_sc as plsc

NC = 2
NS = 16
NW = NC * NS
CH = 128


GRP = 8


def _sc_aggregate(node_embedding, edge3, rid3):
    N = node_embedding.shape[0]
    NP = ((N + NS * 8 - 1) // (NS * 8)) * (NS * 8)
    TOT = edge3.shape[1]
    RCH = rid3.shape[0]
    R = RCH * CH
    NG = TOT // GRP
    GPW = pl.cdiv(NG, NW)
    ROWS_PT = NP // NS
    TPG = RCH // (NS * GRP)

    f32 = jnp.float32
    ne_pad = jnp.pad(node_embedding, ((0, NP - N), (0, 0)))
    zeros8 = jnp.zeros((ROWS_PT, 8), f32)
    ones8 = jnp.ones((CH, 8), f32)

    mesh = plsc.VectorSubcoreMesh(
        core_axis_name="c", subcore_axis_name="s",
        num_cores=NC, num_subcores=NS)

    @functools.partial(
        pl.kernel,
        out_type=[
            jax.ShapeDtypeStruct((NC, R, 8), f32),
            jax.ShapeDtypeStruct((NC, R, 8), f32),
            jax.ShapeDtypeStruct((R, 8), f32),
        ],
        mesh=mesh,
        compiler_params=pltpu.CompilerParams(use_tc_tiling_on_sc=False),
        scratch_types=[
            pltpu.VMEM_SHARED((NP, 8), f32),
            pltpu.VMEM((2, GRP, CH), jnp.int32),
            pltpu.VMEM((CH, 8), f32),
            pltpu.VMEM((CH, 8), f32),
            pltpu.SemaphoreType.DMA,
            pltpu.SemaphoreType.DMA,
            pltpu.SemaphoreType.DMA,
            pltpu.SemaphoreType.DMA,
        ],
    )
    def k(ne_hbm, edge_hbm, rid_hbm, z_hbm, o_hbm,
          aggsel, acc2sel, nselo,
          acc_sh, ibuf, rows0, rows1, sg0, sg1, ss0, ss1):
        cid = lax.axis_index("c")
        sid = lax.axis_index("s")
        w = sid * NC + cid
        rows = (rows0, rows1)
        semg = (sg0, sg1)
        sems = (ss0, ss1)

        r0 = sid * ROWS_PT
        pltpu.sync_copy(z_hbm, acc_sh.at[pl.ds(r0, ROWS_PT)])
        plsc.subcore_barrier()

        g_lo = w * GPW
        g_hi = jnp.minimum(g_lo + GPW, NG)

        def bodyA(g, carry):
            pltpu.sync_copy(edge_hbm.at[:, pl.ds(g * GRP, GRP)], ibuf)
            gd = [None] * GRP
            sd = [None] * GRP
            gd[0] = pltpu.async_copy(ne_hbm.at[ibuf.at[0, 0]], rows0, sg0)
            for j in range(GRP):
                gd[j].wait()
                sd[j] = pltpu.async_copy(
                    rows[j % 2], acc_sh.at[ibuf.at[1, j]], sems[j % 2], add=True)
                if j + 1 < GRP:
                    if j >= 1:
                        sd[j - 1].wait()
                    gd[j + 1] = pltpu.async_copy(
                        ne_hbm.at[ibuf.at[0, j + 1]], rows[(j + 1) % 2],
                        semg[(j + 1) % 2])
            sd[GRP - 2].wait()
            sd[GRP - 1].wait()
            return carry

        lax.fori_loop(g_lo, g_hi, bodyA, 0)
        plsc.subcore_barrier()

        def select(out_ref, src_table, also_nsel):
            for t in range(TPG):
                g = sid * TPG + t
                pltpu.sync_copy(rid_hbm.at[pl.ds(g * GRP, GRP)], ibuf.at[0])
                for j in range(GRP):
                    base = (g * GRP + j) * CH
                    pltpu.async_copy(src_table.at[ibuf.at[0, j]], rows0, sg0).wait()
                    pltpu.sync_copy(rows0, out_ref.at[cid, pl.ds(base, CH)])
                    if also_nsel:
                        @pl.when(cid == (t % NC))
                        def _():
                            pltpu.async_copy(
                                ne_hbm.at[ibuf.at[0, j]], rows1, sg1).wait()
                            pltpu.sync_copy(rows1, nselo.at[pl.ds(base, CH)])

        select(aggsel, acc_sh, True)
        plsc.subcore_barrier()

        pltpu.sync_copy(o_hbm, rows0)

        def bodyB(g, carry):
            pltpu.sync_copy(edge_hbm.at[1, pl.ds(g * GRP, GRP)], ibuf.at[1])
            sd = [pltpu.async_copy(
                      rows0, acc_sh.at[ibuf.at[1, j]], sems[j % 2], add=True)
                  for j in range(GRP)]
            for d in sd:
                d.wait()
            return carry

        lax.fori_loop(g_lo, g_hi, bodyB, 0)
        plsc.subcore_barrier()

        select(acc2sel, acc_sh, False)

    return k(ne_pad, edge3, rid3, zeros8, ones8)


def _tc_bert(bert, W1t, b1r):
    Bn = bert.shape[0]
    BLK = 1024

    def body(bert_ref, W1t_ref, b1_ref, out_ref):
        x1 = jnp.dot(bert_ref[...], W1t_ref[...],
                     preferred_element_type=jnp.float32)
        out_ref[...] = jnp.maximum(x1 + b1_ref[...], 0.0)

    return pl.pallas_call(
        body,
        grid=(Bn // BLK,),
        in_specs=[
            pl.BlockSpec((BLK, 768), lambda i: (i, 0)),
            pl.BlockSpec((768, 128), lambda i: (0, 0)),
            pl.BlockSpec((1, 128), lambda i: (0, 0)),
        ],
        out_specs=pl.BlockSpec((BLK, 128), lambda i: (i, 0)),
        out_shape=jax.ShapeDtypeStruct((Bn, 128), jnp.float32),
    )(bert, W1t, b1r)


def _tc_head(x1, aggsel, acc2sel, nsel,
             Wst, Wnt, bsr, W2a, W2b, W2c, b2r, W3p, b3p):
    Bn = x1.shape[0]
    BLK = 1024
    H = Bn // BLK

    def body(x1_ref, ah_ref, at_ref, ch_ref, ct_ref,
             nh_ref, nt_ref, Wst_ref, Wnt_ref, bs_ref,
             W2a_ref, W2b_ref, W2c_ref, b2_ref, W3p_ref, b3_ref, out_ref):
        f32 = jnp.float32
        ah = ah_ref[0] + ah_ref[1]
        at = at_ref[0] + at_ref[1]
        dh = (ch_ref[0, :, 0:1] + ch_ref[1, :, 0:1]) - ah[:, 0:1]
        dt = (ct_ref[0, :, 0:1] + ct_ref[1, :, 0:1]) - at[:, 0:1]
        hh = ah / jnp.maximum(dh, 1.0)
        ht = at / jnp.maximum(dt, 1.0)
        rel_h = (jnp.dot(nh_ref[...], Wst_ref[...], preferred_element_type=f32)
                 + jnp.dot(hh, Wnt_ref[...], preferred_element_type=f32)
                 + bs_ref[...])
        rel_t = (jnp.dot(nt_ref[...], Wst_ref[...], preferred_element_type=f32)
                 + jnp.dot(ht, Wnt_ref[...], preferred_element_type=f32)
                 + bs_ref[...])
        y = (jnp.dot(x1_ref[...], W2a_ref[...], preferred_element_type=f32)
             + jnp.dot(rel_h, W2b_ref[...], preferred_element_type=f32)
             + jnp.dot(rel_t, W2c_ref[...], preferred_element_type=f32)
             + b2_ref[...])
        y = jnp.maximum(y, 0.0)
        out_ref[...] = jnp.dot(y, W3p_ref[...], preferred_element_type=f32) + b3_ref[...]

    return pl.pallas_call(
        body,
        grid=(Bn // BLK,),
        in_specs=[
            pl.BlockSpec((BLK, 128), lambda i: (i, 0)),
            pl.BlockSpec((2, BLK, 8), lambda i: (0, i, 0)),
            pl.BlockSpec((2, BLK, 8), lambda i: (0, H + i, 0)),
            pl.BlockSpec((2, BLK, 8), lambda i: (0, i, 0)),
            pl.BlockSpec((2, BLK, 8), lambda i: (0, H + i, 0)),
            pl.BlockSpec((BLK, 8), lambda i: (i, 0)),
            pl.BlockSpec((BLK, 8), lambda i: (H + i, 0)),
            pl.BlockSpec((8, 8), lambda i: (0, 0)),
            pl.BlockSpec((8, 8), lambda i: (0, 0)),
            pl.BlockSpec((1, 8), lambda i: (0, 0)),
            pl.BlockSpec((128, 16), lambda i: (0, 0)),
            pl.BlockSpec((8, 16), lambda i: (0, 0)),
            pl.BlockSpec((8, 16), lambda i: (0, 0)),
            pl.BlockSpec((1, 16), lambda i: (0, 0)),
            pl.BlockSpec((16, 8), lambda i: (0, 0)),
            pl.BlockSpec((1, 8), lambda i: (0, 0)),
        ],
        out_specs=pl.BlockSpec((BLK, 8), lambda i: (i, 0)),
        out_shape=jax.ShapeDtypeStruct((Bn, 8), jnp.float32),
    )(x1, aggsel, aggsel, acc2sel, acc2sel, nsel, nsel,
      Wst, Wnt, bsr, W2a, W2b, W2c, b2r, W3p, b3p)


def kernel(bert_embedding, features, row_ids, node_embedding, edge_index,
           W_self, W_neigh, b_sage, W1, b1, W2, b2, W3, b3):
    del features
    E = edge_index.shape[1]
    Bn = bert_embedding.shape[0]

    edge3 = edge_index.astype(jnp.int32).reshape(2, E // CH, CH)
    rid = row_ids.astype(jnp.int32).reshape(-1)
    rid3 = rid.reshape(rid.shape[0] // CH, CH)

    aggsel, acc2sel, nsel = _sc_aggregate(node_embedding, edge3, rid3)
    x1 = _tc_bert(bert_embedding, W1.T, b1.reshape(1, 128))

    Wst = W_self.T
    Wnt = W_neigh.T
    bsr = b_sage.reshape(1, 8)
    W2a = W2[:, :128].T
    W2b = W2[:, 128:136].T
    W2c = W2[:, 136:144].T
    b2r = b2.reshape(1, 16)
    W3p = jnp.pad(W3.T, ((0, 0), (0, 5)))
    b3p = jnp.pad(b3, (0, 5)).reshape(1, 8)

    out8 = _tc_head(x1, aggsel, acc2sel, nsel,
                    Wst, Wnt, bsr, W2a, W2b, W2c, b2r, W3p, b3p)
    return out8[:, :3]
